# baseline (speedup 1.0000x reference)
import functools
import math

import jax
import jax.numpy as jnp
from jax import lax
from jax.experimental import pallas as pl
from jax.experimental.pallas import tpu as pltpu
from jax.experimental.pallas import tpu_sc as plsc

D_MODEL = 64
SCALE = math.sqrt(D_MODEL)
NUM_CORES = 2
NUM_SUBCORES = 16
NUM_WORKERS = NUM_CORES * NUM_SUBCORES
SEQ = 200
BTOK = 128
CHUNK = 512
NBUF = 2


def _gather_body(x_hbm, table_hbm, out_hbm, *scratch, b_per_w):
    idx_v = scratch[:NBUF]
    rows_v = scratch[NBUF:2 * NBUF]
    gsem = scratch[2 * NBUF:3 * NBUF]
    ssem = scratch[3 * NBUF:4 * NBUF]

    wid = lax.axis_index("s") * NUM_CORES + lax.axis_index("c")
    base = wid * b_per_w
    n_chunks = b_per_w // CHUNK

    for b in range(NBUF):
        off = base + b * CHUNK
        pltpu.sync_copy(x_hbm.at[pl.ds(off, CHUNK)], idx_v[b])
        pltpu.async_copy(table_hbm.at[idx_v[b]], rows_v[b], gsem[b])

    def super_body(k, carry):
        for b in range(NBUF):
            cur = k * NBUF + b
            off = base + cur * CHUNK
            pltpu.make_async_copy(table_hbm.at[idx_v[b]], rows_v[b],
                                  gsem[b]).wait()
            pltpu.async_copy(rows_v[b], out_hbm.at[pl.ds(off, CHUNK)],
                             ssem[b])
            nxt = cur + NBUF

            @pl.when(nxt < n_chunks)
            def _():
                noff = base + nxt * CHUNK
                pltpu.sync_copy(x_hbm.at[pl.ds(noff, CHUNK)], idx_v[b])
                pltpu.make_async_copy(
                    rows_v[b], out_hbm.at[pl.ds(off, CHUNK)], ssem[b]).wait()
                pltpu.async_copy(table_hbm.at[idx_v[b]], rows_v[b], gsem[b])

        return carry

    lax.fori_loop(0, n_chunks // NBUF, super_body, 0)

    for b in range(NBUF):
        off = base + (n_chunks - NBUF + b) * CHUNK
        pltpu.make_async_copy(rows_v[b], out_hbm.at[pl.ds(off, CHUNK)],
                              ssem[b]).wait()


def _format_tc(g3, n_rows):
    ngrp = n_rows // BTOK

    def body(g_ref, o_ref):
        blk = g_ref[...].reshape(BTOK, D_MODEL)
        tr = jnp.transpose(blk, (1, 0)) * SCALE
        o_ref[...] = tr.reshape(1, D_MODEL // 8, 1, 8, BTOK)

    return pl.pallas_call(
        body,
        out_shape=jax.ShapeDtypeStruct(
            (SEQ, D_MODEL // 8, ngrp, 8, BTOK), jnp.float32),
        grid=(SEQ, ngrp),
        in_specs=[pl.BlockSpec((1, BTOK, D_MODEL), lambda t, bg: (t, bg, 0))],
        out_specs=pl.BlockSpec((1, D_MODEL // 8, 1, 8, BTOK),
                               lambda t, bg: (t, 0, bg, 0, 0)),
    )(g3)


def kernel(x, table):
    n_rows, seq = x.shape
    b = x.size
    assert seq == SEQ and b % (NUM_WORKERS * CHUNK * NBUF) == 0
    b_per_w = b // NUM_WORKERS

    xt_flat = x.T.reshape(b)

    mesh = plsc.VectorSubcoreMesh(
        core_axis_name="c", subcore_axis_name="s",
        num_cores=NUM_CORES, num_subcores=NUM_SUBCORES,
    )
    scratch = (
        [pltpu.VMEM((CHUNK,), jnp.int32) for _ in range(NBUF)]
        + [pltpu.VMEM((CHUNK, D_MODEL), jnp.float32) for _ in range(NBUF)]
        + [pltpu.SemaphoreType.DMA for _ in range(2 * NBUF)]
    )
    f = functools.partial(
        pl.kernel,
        out_type=jax.ShapeDtypeStruct((b, D_MODEL), jnp.float32),
        mesh=mesh,
        scratch_types=scratch,
        compiler_params=pltpu.CompilerParams(use_tc_tiling_on_sc=False),
    )(functools.partial(_gather_body, b_per_w=b_per_w))
    g = f(xt_flat, table)

    g3 = g.reshape(SEQ, n_rows, D_MODEL)
    out5 = _format_tc(g3, n_rows)
    return out5.transpose(2, 4, 0, 1, 3).reshape(n_rows, SEQ, D_MODEL)


# speedup vs baseline: 3.6736x; 3.6736x over previous
import functools
import math

import jax
import jax.numpy as jnp
from jax import lax
from jax.experimental import pallas as pl
from jax.experimental.pallas import tpu as pltpu
from jax.experimental.pallas import tpu_sc as plsc

D_MODEL = 64
SCALE = math.sqrt(D_MODEL)
NUM_CORES = 2
NUM_SUBCORES = 16
NUM_WORKERS = NUM_CORES * NUM_SUBCORES
LANES = 16
CHUNK = 512
NBUF = 2


def _emb_body(x_hbm, table_hbm, out_hbm, *scratch, b_per_w):
    idx_v = scratch[:NBUF]
    rows_v = scratch[NBUF:2 * NBUF]
    gsem = scratch[2 * NBUF:3 * NBUF]
    ssem = scratch[3 * NBUF:4 * NBUF]

    wid = lax.axis_index("s") * NUM_CORES + lax.axis_index("c")
    base = wid * b_per_w
    n_chunks = b_per_w // CHUNK

    for b in range(NBUF):
        off = base + b * CHUNK
        pltpu.sync_copy(x_hbm.at[pl.ds(off, CHUNK)], idx_v[b])
        pltpu.async_copy(table_hbm.at[idx_v[b]], rows_v[b], gsem[b])

    def super_body(k, carry):
        for b in range(NBUF):
            cur = k * NBUF + b
            off = base + cur * CHUNK
            pltpu.make_async_copy(table_hbm.at[idx_v[b]], rows_v[b],
                                  gsem[b]).wait()

            @plsc.parallel_loop(0, CHUNK, step=1, unroll=8)
            def _mul(i):
                for j in range(D_MODEL // LANES):
                    sl = pl.ds(j * LANES, LANES)
                    rows_v[b][i, sl] = rows_v[b][i, sl] * SCALE

            pltpu.async_copy(rows_v[b], out_hbm.at[pl.ds(off, CHUNK)],
                             ssem[b])
            nxt = cur + NBUF

            @pl.when(nxt < n_chunks)
            def _():
                noff = base + nxt * CHUNK
                pltpu.sync_copy(x_hbm.at[pl.ds(noff, CHUNK)], idx_v[b])
                pltpu.make_async_copy(
                    rows_v[b], out_hbm.at[pl.ds(off, CHUNK)], ssem[b]).wait()
                pltpu.async_copy(table_hbm.at[idx_v[b]], rows_v[b], gsem[b])

        return carry

    lax.fori_loop(0, n_chunks // NBUF, super_body, 0)

    for b in range(NBUF):
        off = base + (n_chunks - NBUF + b) * CHUNK
        pltpu.make_async_copy(rows_v[b], out_hbm.at[pl.ds(off, CHUNK)],
                              ssem[b]).wait()


def kernel(x, table):
    orig_shape = x.shape
    b = x.size
    assert b % (NUM_WORKERS * CHUNK * NBUF) == 0
    b_per_w = b // NUM_WORKERS

    x_flat = x.T.reshape(b)

    mesh = plsc.VectorSubcoreMesh(
        core_axis_name="c", subcore_axis_name="s",
        num_cores=NUM_CORES, num_subcores=NUM_SUBCORES,
    )
    scratch = (
        [pltpu.VMEM((CHUNK,), jnp.int32) for _ in range(NBUF)]
        + [pltpu.VMEM((CHUNK, D_MODEL), jnp.float32) for _ in range(NBUF)]
        + [pltpu.SemaphoreType.DMA for _ in range(2 * NBUF)]
    )
    f = functools.partial(
        pl.kernel,
        out_type=jax.ShapeDtypeStruct((b, D_MODEL), jnp.float32),
        mesh=mesh,
        scratch_types=scratch,
        compiler_params=pltpu.CompilerParams(use_tc_tiling_on_sc=False),
    )(functools.partial(_emb_body, b_per_w=b_per_w))
    out = f(x_flat, table)
    return out.reshape(orig_shape[1], orig_shape[0],
                       D_MODEL).transpose(1, 0, 2)
